# baseline (speedup 1.0000x reference)
import jax
import jax.numpy as jnp
from jax import lax
from jax.experimental import pallas as pl
from jax.experimental.pallas import tpu as pltpu
from jax.experimental.pallas import tpu_sc as plsc

_EMB = 64
_CLASSES = 1000
_B = 1024
_S = 50
_L = 20
_CPB = _S
_NW = 32
_BPW = _B // _NW
_LP = 32
_TPW = _BPW * _S * _LP
_CPW = _BPW * _CPB
_DEPTH = 8
_NT = _B // 128
_LPW = _L // 4
_WIN = _S * 128


def _detile_body(tokst_hbm, out_hbm, sent, outb, sems):
    c = lax.axis_index("c")
    s = lax.axis_index("s")
    wid = s * 2 + c
    t = wid & 7
    l0 = (wid >> 3) * _LPW

    def start(j, q):
        pltpu.async_copy(
            tokst_hbm.at[l0 + j, :, pl.ds(128 * t, 128)], sent[q], sems[q]
        )

    def wait(q):
        pltpu.make_async_copy(
            tokst_hbm.at[0, :, pl.ds(0, 128)], sent[q], sems[q]
        ).wait()

    start(0, 0)
    for j in range(_LPW):
        q = j % 2
        if j + 1 < _LPW:
            start(j + 1, 1 - q)
        wait(q)
        for k in range(_S):
            for k2 in range(8):
                outb[pl.ds(k * 128 + 16 * k2, 16)] = sent[q][
                    k, pl.ds(16 * k2, 16)
                ]
        pltpu.sync_copy(
            outb, out_hbm.at[pl.ds((t * _L + l0 + j) * _WIN, _WIN)]
        )


def _detile_toks(tokst):
    k = pl.kernel(
        _detile_body,
        out_type=jax.ShapeDtypeStruct((_B * _S * _L,), jnp.int32),
        mesh=plsc.VectorSubcoreMesh(core_axis_name="c", subcore_axis_name="s"),
        scratch_types=[
            [pltpu.VMEM((_S, 128), jnp.int32)] * 2,
            pltpu.VMEM((_WIN,), jnp.int32),
            [pltpu.SemaphoreType.DMA] * 2,
        ],
    )
    return k(tokst)


def _emb_body(table_hbm, flatt_hbm, out_hbm, slab_v, idx_v, rows, stage_v,
              sems, ssem):
    c = lax.axis_index("c")
    s = lax.axis_index("s")
    wid = s * 2 + c
    t = wid >> 2
    sub = wid & 3
    iota = lax.iota(jnp.int32, 16) * (_S * _LP)

    half_words = _L // 2 * _WIN

    for half in range(2):
        pltpu.sync_copy(
            flatt_hbm.at[pl.ds(t * _L * _WIN + half * half_words, half_words)],
            slab_v,
        )

        def lrow(li, carry):
            l = half * (_L // 2) + li
            for k in range(_S):
                for h in range(2):
                    off = (li * _S + k) * 128 + sub * _LP + 16 * h
                    v = slab_v[pl.ds(off, 16)]
                    addr = iota + (16 * h * _S * _LP + k * _LP + l)
                    plsc.store_scatter(idx_v, [addr], v)
            return carry

        lax.fori_loop(0, _L // 2, lrow, 0)

    def start(g, q):
        pltpu.async_copy(
            table_hbm.at[idx_v.at[pl.ds(g * _LP, _L)]], rows[q], sems[q]
        )

    def wait(q):
        pltpu.make_async_copy(
            table_hbm.at[idx_v.at[pl.ds(0, _L)]], rows[q], sems[q]
        ).wait()

    def accumulate(q, accs):
        accs = list(accs)
        for r in range(_L):
            p = (r % 2) * 4
            for cb in range(4):
                accs[p + cb] = accs[p + cb] + rows[q][r, pl.ds(16 * cb, 16)]
        return accs

    def boundary(g, accs):
        bnd = lax.rem(g, _CPB) == _CPB - 1
        i = lax.div(g, _CPB)

        @pl.when(bnd)
        def _():
            for cb in range(4):
                stage_v[i, pl.ds(16 * cb, 16)] = accs[cb] + accs[4 + cb]

        zero = jnp.zeros((16,), jnp.float32)
        return tuple(jnp.where(bnd, zero, a) for a in accs)

    for q in range(_DEPTH):
        start(q, q)

    def round_body(j, accs):
        for q in range(_DEPTH):
            g = _DEPTH * j + q
            wait(q)
            accs = accumulate(q, accs)
            accs = boundary(g, accs)

            @pl.when(g + _DEPTH < _CPW)
            def _():
                start(g + _DEPTH, q)

        return tuple(accs)

    zero = jnp.zeros((16,), jnp.float32)
    lax.fori_loop(0, _CPW // _DEPTH, round_body, (zero,) * 8)
    pltpu.sync_copy(stage_v, out_hbm.at[pl.ds(wid * _BPW, _BPW)])


def _embedding_sums(table, flatt):
    k = pl.kernel(
        _emb_body,
        out_type=jax.ShapeDtypeStruct((_B, _EMB), jnp.float32),
        mesh=plsc.VectorSubcoreMesh(core_axis_name="c", subcore_axis_name="s"),
        compiler_params=pltpu.CompilerParams(
            use_tc_tiling_on_sc=False, needs_layout_passes=False
        ),
        scratch_types=[
            pltpu.VMEM((_L // 2 * _WIN,), jnp.int32),
            pltpu.VMEM((_TPW,), jnp.int32),
            [pltpu.VMEM((_L, _EMB), jnp.float32)] * _DEPTH,
            pltpu.VMEM((_BPW, _EMB), jnp.float32),
            [pltpu.SemaphoreType.DMA] * _DEPTH,
            pltpu.SemaphoreType.DMA,
        ],
    )
    return k(table, flatt)


def _linear_body(x_ref, w_ref, b_ref, o_ref):
    x = x_ref[...] * (1.0 / (_S * _L))
    o_ref[...] = (
        jnp.dot(x, w_ref[...], preferred_element_type=jnp.float32) + b_ref[...]
    )


def _linear(sums, wt, b2):
    return pl.pallas_call(
        _linear_body,
        out_shape=jax.ShapeDtypeStruct((_B, _CLASSES), jnp.float32),
    )(sums, wt, b2)


def kernel(tok_lists_batch, table, W, b):
    tokst = jnp.transpose(tok_lists_batch.astype(jnp.int32), (2, 1, 0))
    flatt = _detile_toks(tokst)
    sums = _embedding_sums(table, flatt)
    return _linear(sums, W.T, b.reshape(1, _CLASSES))


# speedup vs baseline: 1.0201x; 1.0201x over previous
import jax
import jax.numpy as jnp
from jax import lax
from jax.experimental import pallas as pl
from jax.experimental.pallas import tpu as pltpu
from jax.experimental.pallas import tpu_sc as plsc

_EMB = 64
_CLASSES = 1000
_B = 1024
_S = 50
_L = 20
_CPB = _S
_NW = 32
_BPW = _B // _NW
_LP = 32
_TPW = _BPW * _S * _LP
_CPW = _BPW * _CPB
_DEPTH = 8
_NT = _B // 128
_LPW = _L // 4
_WIN = _S * 128
_VOCAB = 1_000_000
_PW = _EMB // 2
_PCHUNK = 256
_NFULL = _VOCAB // _PCHUNK
_PTAIL = _VOCAB - _NFULL * _PCHUNK
_PROUND = -(-_NFULL // _NW)


def _detile_body(tokst_hbm, out_hbm, sent, outb, sems):
    c = lax.axis_index("c")
    s = lax.axis_index("s")
    wid = s * 2 + c
    t = wid & 7
    l0 = (wid >> 3) * _LPW

    def start(j, q):
        pltpu.async_copy(
            tokst_hbm.at[l0 + j, :, pl.ds(128 * t, 128)], sent[q], sems[q]
        )

    def wait(q):
        pltpu.make_async_copy(
            tokst_hbm.at[0, :, pl.ds(0, 128)], sent[q], sems[q]
        ).wait()

    start(0, 0)
    for j in range(_LPW):
        q = j % 2
        if j + 1 < _LPW:
            start(j + 1, 1 - q)
        wait(q)
        for k in range(_S):
            for k2 in range(8):
                outb[pl.ds(k * 128 + 16 * k2, 16)] = sent[q][
                    k, pl.ds(16 * k2, 16)
                ]
        pltpu.sync_copy(
            outb, out_hbm.at[pl.ds((t * _L + l0 + j) * _WIN, _WIN)]
        )


def _detile_toks(tokst):
    k = pl.kernel(
        _detile_body,
        out_type=jax.ShapeDtypeStruct((_B * _S * _L,), jnp.int32),
        mesh=plsc.VectorSubcoreMesh(core_axis_name="c", subcore_axis_name="s"),
        scratch_types=[
            [pltpu.VMEM((_S, 128), jnp.int32)] * 2,
            pltpu.VMEM((_WIN,), jnp.int32),
            [pltpu.SemaphoreType.DMA] * 2,
        ],
    )
    return k(tokst)


def _pack_rows(src, dst, nrows):
    half = jnp.int32(0x8000)
    himask = jnp.int32(-65536)

    def row_block(ri, carry):
        for u in range(8):
            r = ri * 8 + u
            ws = []
            for grp in range(2):
                a = lax.bitcast_convert_type(
                    src[r, pl.ds(32 * grp, 16)], jnp.int32
                )
                bq = lax.bitcast_convert_type(
                    src[r, pl.ds(32 * grp + 16, 16)], jnp.int32
                )
                lo = lax.shift_right_logical(a + half, 16)
                hi = (bq + half) & himask
                ws.append(lo | hi)
            dst[pl.ds(r * _PW, 16)] = ws[0]
            dst[pl.ds(r * _PW + 16, 16)] = ws[1]
        return carry

    lax.fori_loop(0, nrows // 8, row_block, 0)


def _pack_body(table_hbm, out_hbm, ins, outs, lsems, ssems):
    c = lax.axis_index("c")
    s = lax.axis_index("s")
    wid = s * 2 + c

    def start_load(g, q):
        pltpu.async_copy(
            table_hbm.at[pl.ds(g * _PCHUNK, _PCHUNK)], ins[q], lsems[q]
        )

    def wait_load(q):
        pltpu.make_async_copy(
            table_hbm.at[pl.ds(0, _PCHUNK)], ins[q], lsems[q]
        ).wait()

    def start_store(g, q):
        pltpu.async_copy(
            outs[q], out_hbm.at[pl.ds(g * _PCHUNK * _PW, _PCHUNK * _PW)],
            ssems[q],
        )

    def wait_store(q):
        pltpu.make_async_copy(
            outs[q], out_hbm.at[pl.ds(0, _PCHUNK * _PW)], ssems[q]
        ).wait()

    @pl.when(wid < _NFULL)
    def _():
        start_load(wid, 0)

    def round_body(j, carry):
        for q in range(2):
            jj = 2 * j + q
            g = wid + jj * _NW
            nxt = g + _NW

            @pl.when(nxt < _NFULL)
            def _():
                start_load(nxt, 1 - q)

            @pl.when(g < _NFULL)
            def _():
                wait_load(q)

                @pl.when(jj >= 2)
                def _():
                    wait_store(q)

                _pack_rows(ins[q], outs[q], _PCHUNK)
                start_store(g, q)

        return carry

    lax.fori_loop(0, (_PROUND + 1) // 2, round_body, 0)

    for q in range(2):
        wait_store(q)

    @pl.when(wid == 0)
    def _():
        pltpu.async_copy(
            table_hbm.at[pl.ds(_NFULL * _PCHUNK, _PTAIL)],
            ins[0].at[pl.ds(0, _PTAIL)],
            lsems[0],
        )
        pltpu.make_async_copy(
            table_hbm.at[pl.ds(0, _PTAIL)],
            ins[0].at[pl.ds(0, _PTAIL)],
            lsems[0],
        ).wait()
        _pack_rows(ins[0], outs[0], _PTAIL)
        pltpu.async_copy(
            outs[0].at[pl.ds(0, _PTAIL * _PW)],
            out_hbm.at[pl.ds(_NFULL * _PCHUNK * _PW, _PTAIL * _PW)],
            ssems[0],
        )
        pltpu.make_async_copy(
            outs[0].at[pl.ds(0, _PTAIL * _PW)],
            out_hbm.at[pl.ds(0, _PTAIL * _PW)],
            ssems[0],
        ).wait()


def _pack_table(table):
    k = pl.kernel(
        _pack_body,
        out_type=jax.ShapeDtypeStruct((_VOCAB * _PW,), jnp.int32),
        mesh=plsc.VectorSubcoreMesh(core_axis_name="c", subcore_axis_name="s"),
        scratch_types=[
            [pltpu.VMEM((_PCHUNK, _EMB), jnp.float32)] * 2,
            [pltpu.VMEM((_PCHUNK * _PW,), jnp.int32)] * 2,
            [pltpu.SemaphoreType.DMA] * 2,
            [pltpu.SemaphoreType.DMA] * 2,
        ],
    )
    return k(table)


def _emb_body(tp_hbm, flatt_hbm, out_hbm, slab_v, idx_v, rows, stage_v,
              sems, ssem):
    c = lax.axis_index("c")
    s = lax.axis_index("s")
    wid = s * 2 + c
    t = wid >> 2
    sub = wid & 3
    iota = lax.iota(jnp.int32, 16) * (_S * _LP)

    half_words = _L // 2 * _WIN

    for half in range(2):
        pltpu.sync_copy(
            flatt_hbm.at[pl.ds(t * _L * _WIN + half * half_words, half_words)],
            slab_v,
        )

        def lrow(li, carry):
            l = half * (_L // 2) + li
            for k in range(_S):
                for h in range(2):
                    off = (li * _S + k) * 128 + sub * _LP + 16 * h
                    v = slab_v[pl.ds(off, 16)]
                    addr = iota + (16 * h * _S * _LP + k * _LP + l)
                    plsc.store_scatter(idx_v, [addr], v)
            return carry

        lax.fori_loop(0, _L // 2, lrow, 0)

    def start(g, q):
        pltpu.async_copy(
            tp_hbm.at[idx_v.at[pl.ds(g * _LP, _L)]], rows[q], sems[q]
        )

    def wait(q):
        pltpu.make_async_copy(
            tp_hbm.at[idx_v.at[pl.ds(0, _L)]], rows[q], sems[q]
        ).wait()

    himask = jnp.int32(-65536)

    def accumulate(q, accs):
        accs = list(accs)
        for r in range(_L):
            p = (r % 2) * 4
            for grp in range(2):
                w = rows[q][r, pl.ds(16 * grp, 16)]
                lo = lax.bitcast_convert_type(
                    lax.shift_left(w, 16), jnp.float32
                )
                hi = lax.bitcast_convert_type(w & himask, jnp.float32)
                accs[p + 2 * grp] = accs[p + 2 * grp] + lo
                accs[p + 2 * grp + 1] = accs[p + 2 * grp + 1] + hi
        return accs

    def boundary(g, accs):
        bnd = lax.rem(g, _CPB) == _CPB - 1
        i = lax.div(g, _CPB)

        @pl.when(bnd)
        def _():
            for cb in range(4):
                stage_v[i, pl.ds(16 * cb, 16)] = accs[cb] + accs[4 + cb]

        zero = jnp.zeros((16,), jnp.float32)
        return tuple(jnp.where(bnd, zero, a) for a in accs)

    for q in range(_DEPTH):
        start(q, q)

    def round_body(j, accs):
        for q in range(_DEPTH):
            g = _DEPTH * j + q
            wait(q)
            accs = accumulate(q, accs)
            accs = boundary(g, accs)

            @pl.when(g + _DEPTH < _CPW)
            def _():
                start(g + _DEPTH, q)

        return tuple(accs)

    zero = jnp.zeros((16,), jnp.float32)
    lax.fori_loop(0, _CPW // _DEPTH, round_body, (zero,) * 8)
    pltpu.sync_copy(stage_v, out_hbm.at[pl.ds(wid * _BPW, _BPW)])


def _embedding_sums(tp, flatt):
    k = pl.kernel(
        _emb_body,
        out_type=jax.ShapeDtypeStruct((_B, _EMB), jnp.float32),
        mesh=plsc.VectorSubcoreMesh(core_axis_name="c", subcore_axis_name="s"),
        compiler_params=pltpu.CompilerParams(
            use_tc_tiling_on_sc=False, needs_layout_passes=False
        ),
        scratch_types=[
            pltpu.VMEM((_L // 2 * _WIN,), jnp.int32),
            pltpu.VMEM((_TPW,), jnp.int32),
            [pltpu.VMEM((_L, _PW), jnp.int32)] * _DEPTH,
            pltpu.VMEM((_BPW, _EMB), jnp.float32),
            [pltpu.SemaphoreType.DMA] * _DEPTH,
            pltpu.SemaphoreType.DMA,
        ],
    )
    return k(tp, flatt)


def _linear_body(x_ref, w_ref, b_ref, o_ref):
    x = x_ref[...] * (1.0 / (_S * _L))
    o_ref[...] = (
        jnp.dot(x, w_ref[...], preferred_element_type=jnp.float32) + b_ref[...]
    )


def _linear(sums, wt, b2):
    return pl.pallas_call(
        _linear_body,
        out_shape=jax.ShapeDtypeStruct((_B, _CLASSES), jnp.float32),
    )(sums, wt, b2)


def kernel(tok_lists_batch, table, W, b):
    tokst = jnp.transpose(tok_lists_batch.astype(jnp.int32), (2, 1, 0))
    flatt = _detile_toks(tokst)
    tp = _pack_table(table).reshape(_VOCAB, _PW)
    sums = _embedding_sums(tp, flatt)
    return _linear(sums, W.T, b.reshape(1, _CLASSES))


# speedup vs baseline: 1.0257x; 1.0055x over previous
import jax
import jax.numpy as jnp
from jax import lax
from jax.experimental import pallas as pl
from jax.experimental.pallas import tpu as pltpu
from jax.experimental.pallas import tpu_sc as plsc

_EMB = 64
_CLASSES = 1000
_B = 1024
_S = 50
_L = 20
_CPB = _S
_NW = 32
_BPW = _B // _NW
_LP = 32
_TPW = _BPW * _S * _LP
_CPW = _BPW * _CPB
_DEPTH = 8


def _flatten_body(toks_hbm, out_hbm, sent, flat_v, sems):
    c = lax.axis_index("c")
    s = lax.axis_index("s")
    wid = s * 2 + c

    def start(b, q):
        pltpu.async_copy(toks_hbm.at[wid * _BPW + b], sent[q], sems[q])

    def wait(q):
        pltpu.make_async_copy(toks_hbm.at[0], sent[q], sems[q]).wait()

    def flatten(b, q):
        for k in range(_S):
            base = (b * _S + k) * _LP
            flat_v[pl.ds(base, 16)] = sent[q][k, pl.ds(0, 16)]
            flat_v[pl.ds(base + 16, 16)] = lax.rev(
                sent[q][k, pl.ds(4, 16)], (0,)
            )

    start(0, 0)

    def pair_body(j, carry):
        ba = 2 * j
        start(ba + 1, 1)
        wait(0)
        flatten(ba, 0)

        @pl.when(ba + 2 < _BPW)
        def _():
            start(ba + 2, 0)

        wait(1)
        flatten(ba + 1, 1)
        return carry

    lax.fori_loop(0, _BPW // 2, pair_body, 0)
    pltpu.sync_copy(flat_v, out_hbm.at[pl.ds(wid * _TPW, _TPW)])


def _flatten_toks(toks):
    k = pl.kernel(
        _flatten_body,
        out_type=jax.ShapeDtypeStruct((_B * _S * _LP,), jnp.int32),
        mesh=plsc.VectorSubcoreMesh(core_axis_name="c", subcore_axis_name="s"),
        scratch_types=[
            [pltpu.VMEM((_S, _L), jnp.int32)] * 2,
            pltpu.VMEM((_TPW,), jnp.int32),
            [pltpu.SemaphoreType.DMA] * 2,
        ],
    )
    return k(toks)


def _emb_body(table_hbm, idx_hbm, out_hbm, idx_v, rows, stage_v, sems):
    c = lax.axis_index("c")
    s = lax.axis_index("s")
    wid = s * 2 + c
    pltpu.sync_copy(idx_hbm.at[pl.ds(wid * _TPW, _TPW)], idx_v)

    def start(g, q):
        pltpu.async_copy(
            table_hbm.at[idx_v.at[pl.ds(g * _LP, _L)]], rows[q], sems[q]
        )

    def wait(q):
        pltpu.make_async_copy(
            table_hbm.at[idx_v.at[pl.ds(0, _L)]], rows[q], sems[q]
        ).wait()

    def accumulate(q, accs):
        accs = list(accs)
        for r in range(_L):
            p = (r % 2) * 4
            for cb in range(4):
                accs[p + cb] = accs[p + cb] + rows[q][r, pl.ds(16 * cb, 16)]
        return accs

    def boundary(g, accs):
        bnd = lax.rem(g, _CPB) == _CPB - 1
        i = lax.div(g, _CPB)

        @pl.when(bnd)
        def _():
            for cb in range(4):
                stage_v[i, pl.ds(16 * cb, 16)] = accs[cb] + accs[4 + cb]

        zero = jnp.zeros((16,), jnp.float32)
        return tuple(jnp.where(bnd, zero, a) for a in accs)

    for q in range(_DEPTH):
        start(q, q)

    def round_body(j, accs):
        for q in range(_DEPTH):
            g = _DEPTH * j + q
            wait(q)
            accs = accumulate(q, accs)
            accs = boundary(g, accs)

            @pl.when(g + _DEPTH < _CPW)
            def _():
                start(g + _DEPTH, q)

        return tuple(accs)

    zero = jnp.zeros((16,), jnp.float32)
    lax.fori_loop(0, _CPW // _DEPTH, round_body, (zero,) * 8)
    pltpu.sync_copy(stage_v, out_hbm.at[pl.ds(wid * _BPW, _BPW)])


def _embedding_sums(table, idx):
    k = pl.kernel(
        _emb_body,
        out_type=jax.ShapeDtypeStruct((_B, _EMB), jnp.float32),
        mesh=plsc.VectorSubcoreMesh(core_axis_name="c", subcore_axis_name="s"),
        compiler_params=pltpu.CompilerParams(use_tc_tiling_on_sc=False),
        scratch_types=[
            pltpu.VMEM((_TPW,), jnp.int32),
            [pltpu.VMEM((_L, _EMB), jnp.float32)] * _DEPTH,
            pltpu.VMEM((_BPW, _EMB), jnp.float32),
            [pltpu.SemaphoreType.DMA] * _DEPTH,
        ],
    )
    return k(table, idx)


def _linear_body(x_ref, w_ref, b_ref, o_ref):
    x = x_ref[...] * (1.0 / (_S * _L))
    o_ref[...] = (
        jnp.dot(x, w_ref[...], preferred_element_type=jnp.float32) + b_ref[...]
    )


def _linear(sums, wt, b2):
    return pl.pallas_call(
        _linear_body,
        out_shape=jax.ShapeDtypeStruct((_B, _CLASSES), jnp.float32),
    )(sums, wt, b2)


def kernel(tok_lists_batch, table, W, b):
    toks = tok_lists_batch.astype(jnp.int32)
    idx = _flatten_toks(toks)
    sums = _embedding_sums(table, idx)
    return _linear(sums, W.T, b.reshape(1, _CLASSES))
